# baseline (speedup 1.0000x reference)
import functools

import jax
import jax.numpy as jnp
from jax import lax
from jax.experimental import pallas as pl
from jax.experimental.pallas import tpu as pltpu
from jax.experimental.pallas import tpu_sc as plsc

BATCH = 16384
EMB = 32
GATHER_WINDOW = 128


def _sc_gather(user_idx, movie_idx, user_table, movie_table):
    b = user_idx.shape[0]
    uidx = user_idx.astype(jnp.int32)
    midx = movie_idx.astype(jnp.int32)
    mesh = plsc.VectorSubcoreMesh(core_axis_name="core", subcore_axis_name="subcore")
    nw = 32
    w = b // nw

    @functools.partial(
        pl.kernel,
        out_type=(
            jax.ShapeDtypeStruct((b, EMB), jnp.float32),
            jax.ShapeDtypeStruct((b, EMB), jnp.float32),
        ),
        mesh=mesh,
        scratch_types=[
            pltpu.VMEM((w,), jnp.int32),
            pltpu.VMEM((w,), jnp.int32),
            pltpu.SemaphoreType.DMA,
            pltpu.SemaphoreType.DMA,
        ],
    )
    def gather_kernel(utab_hbm, mtab_hbm, uidx_hbm, midx_hbm, uout_hbm, mout_hbm,
                      uidx_v, midx_v, sem_i, sem_g):
        wid = lax.axis_index("subcore") * 2 + lax.axis_index("core")
        base = wid * w
        cpu = pltpu.async_copy(uidx_hbm.at[pl.ds(base, w)], uidx_v, sem_i)
        cpm = pltpu.async_copy(midx_hbm.at[pl.ds(base, w)], midx_v, sem_i)
        cpu.wait()
        cpm.wait()

        @pl.loop(0, w, step=16)
        def _(c):
            uvec = uidx_v[pl.ds(c, 16)]
            mvec = midx_v[pl.ds(c, 16)]
            for j in range(16):
                iu = uvec[j]
                im = mvec[j]
                pltpu.async_copy(utab_hbm.at[pl.ds(iu, 1)],
                                 uout_hbm.at[pl.ds(base + c + j, 1)], sem_g)
                pltpu.async_copy(mtab_hbm.at[pl.ds(im, 1)],
                                 mout_hbm.at[pl.ds(base + c + j, 1)], sem_g)

        pltpu.make_async_copy(uout_hbm.at[pl.ds(base, w)],
                              uout_hbm.at[pl.ds(base, w)], sem_g).wait()
        pltpu.make_async_copy(mout_hbm.at[pl.ds(base, w)],
                              mout_hbm.at[pl.ds(base, w)], sem_g).wait()

    return gather_kernel(user_table, movie_table, uidx, midx)


def _mlp_body(u_ref, m_ref, w1u_ref, w1m_ref, b1_ref, w2_ref, b2_ref,
              w3_ref, b3_ref, o_ref):
    dn = (((1,), (1,)), ((), ()))
    hp = jax.lax.Precision.HIGHEST
    u = u_ref[...]
    m = m_ref[...]
    h = lax.dot_general(u, w1u_ref[...], dn, precision=hp,
                        preferred_element_type=jnp.float32)
    h += lax.dot_general(m, w1m_ref[...], dn, precision=hp,
                         preferred_element_type=jnp.float32)
    h = jnp.maximum(h + b1_ref[...][None, :], 0.0)
    h = lax.dot_general(h, w2_ref[...], dn, precision=hp,
                        preferred_element_type=jnp.float32)
    h = jnp.maximum(h + b2_ref[...][None, :], 0.0)
    o_ref[...] = jnp.sum(h * w3_ref[...][0][None, :], axis=1) + b3_ref[...]


def _tc_mlp(u_vec, m_vec, W1, b1, W2, b2, W3, b3):
    b = u_vec.shape[0]
    bm = 2048
    w1u = W1[:, :EMB]
    w1m = W1[:, EMB:]
    grid = (b // bm,)
    return pl.pallas_call(
        _mlp_body,
        grid=grid,
        in_specs=[
            pl.BlockSpec((bm, EMB), lambda i: (i, 0)),
            pl.BlockSpec((bm, EMB), lambda i: (i, 0)),
            pl.BlockSpec(w1u.shape, lambda i: (0, 0)),
            pl.BlockSpec(w1m.shape, lambda i: (0, 0)),
            pl.BlockSpec(b1.shape, lambda i: (0,)),
            pl.BlockSpec(W2.shape, lambda i: (0, 0)),
            pl.BlockSpec(b2.shape, lambda i: (0,)),
            pl.BlockSpec(W3.shape, lambda i: (0, 0)),
            pl.BlockSpec(b3.shape, lambda i: (0,)),
        ],
        out_specs=pl.BlockSpec((bm,), lambda i: (i,)),
        out_shape=jax.ShapeDtypeStruct((b,), jnp.float32),
    )(u_vec, m_vec, w1u, w1m, b1, W2, b2, W3, b3)


def kernel(user_idx, movie_idx, user_table, movie_table, W1, b1, W2, b2, W3, b3):
    u_vec, m_vec = _sc_gather(user_idx, movie_idx, user_table, movie_table)
    return _tc_mlp(u_vec, m_vec, W1, b1, W2, b2, W3, b3)


# speedup vs baseline: 2.2132x; 2.2132x over previous
import functools

import jax
import jax.numpy as jnp
from jax import lax
from jax.experimental import pallas as pl
from jax.experimental.pallas import tpu as pltpu
from jax.experimental.pallas import tpu_sc as plsc

EMB = 32
GRP = 8
NW = 32
CHUNK = 32


def _sc_gather_rows(uidx, midx, utab, mtab):
    b = uidx.shape[0]
    w = b // NW
    c_rows = 128
    nch = w // c_rows
    mesh = plsc.VectorSubcoreMesh(core_axis_name="core", subcore_axis_name="subcore")

    @functools.partial(
        pl.kernel,
        out_type=(
            jax.ShapeDtypeStruct((b, EMB), jnp.float32),
            jax.ShapeDtypeStruct((b, EMB), jnp.float32),
        ),
        mesh=mesh,
        scratch_types=[
            pltpu.VMEM((w,), jnp.int32),
            pltpu.VMEM((w,), jnp.int32),
            pltpu.VMEM((c_rows, EMB), jnp.float32),
            pltpu.VMEM((c_rows, EMB), jnp.float32),
            pltpu.SemaphoreType.DMA,
            pltpu.SemaphoreType.DMA,
            pltpu.SemaphoreType.DMA,
            pltpu.SemaphoreType.DMA,
        ],
    )
    def gather_kernel(utab_hbm, mtab_hbm, uidx_hbm, midx_hbm, uout_hbm, mout_hbm,
                      uidx_v, midx_v, buf0, buf1, sem_i, sem0, sem1, sem_w):
        wid = lax.axis_index("subcore") * 2 + lax.axis_index("core")
        base = wid * w
        cpu = pltpu.async_copy(uidx_hbm.at[pl.ds(base, w)], uidx_v, sem_i)
        cpm = pltpu.async_copy(midx_hbm.at[pl.ds(base, w)], midx_v, sem_i)
        cpu.wait()
        cpm.wait()

        bufs = (buf0, buf1)
        sems = (sem0, sem1)
        for tab_hbm, idx_v, out_hbm in ((utab_hbm, uidx_v, uout_hbm),
                                        (mtab_hbm, midx_v, mout_hbm)):
            wbs = [None, None]
            for c in range(nch):
                p = c & 1
                if wbs[p] is not None:
                    wbs[p].wait()
                    wbs[p] = None
                buf = bufs[p]

                @pl.loop(0, c_rows, step=16)
                def _(cc, _c=c, _buf=buf, _idx_v=idx_v, _tab=tab_hbm, _sem=sems[p]):
                    vec = _idx_v[pl.ds(_c * c_rows + cc, 16)]
                    for j in range(16):
                        i = vec[j]
                        pltpu.async_copy(_tab.at[pl.ds(i, 1)],
                                         _buf.at[pl.ds(cc + j, 1)], _sem)

                pltpu.make_async_copy(
                    out_hbm.at[pl.ds(base + c * c_rows, c_rows)], buf, sems[p]
                ).wait()
                wbs[p] = pltpu.async_copy(
                    buf, out_hbm.at[pl.ds(base + c * c_rows, c_rows)], sem_w)
            for wb in wbs:
                if wb is not None:
                    wb.wait()

    return gather_kernel(utab, mtab, uidx, midx)


def _mlp_body(u_ref, m_ref, w1u_ref, w1m_ref, b1_ref,
              w2_ref, b2_ref, w3_ref, b3_ref, o_ref):
    dn = (((1,), (1,)), ((), ()))
    hp = jax.lax.Precision.HIGHEST
    u = u_ref[...]
    m = m_ref[...]
    h = lax.dot_general(u, w1u_ref[...], dn, precision=hp,
                        preferred_element_type=jnp.float32)
    h += lax.dot_general(m, w1m_ref[...], dn, precision=hp,
                         preferred_element_type=jnp.float32)
    h = jnp.maximum(h + b1_ref[...][None, :], 0.0)
    h = lax.dot_general(h, w2_ref[...], dn, precision=hp,
                        preferred_element_type=jnp.float32)
    h = jnp.maximum(h + b2_ref[...][None, :], 0.0)
    o_ref[...] = jnp.sum(h * w3_ref[...][0][None, :], axis=1) + b3_ref[...]


def _tc_mlp(u_vec, m_vec, W1, b1, W2, b2, W3, b3):
    b = u_vec.shape[0]
    bm = 2048
    w1u = W1[:, :EMB]
    w1m = W1[:, EMB:]
    grid = (b // bm,)
    return pl.pallas_call(
        _mlp_body,
        grid=grid,
        in_specs=[
            pl.BlockSpec((bm, EMB), lambda i: (i, 0)),
            pl.BlockSpec((bm, EMB), lambda i: (i, 0)),
            pl.BlockSpec(w1u.shape, lambda i: (0, 0)),
            pl.BlockSpec(w1m.shape, lambda i: (0, 0)),
            pl.BlockSpec(b1.shape, lambda i: (0,)),
            pl.BlockSpec(W2.shape, lambda i: (0, 0)),
            pl.BlockSpec(b2.shape, lambda i: (0,)),
            pl.BlockSpec(W3.shape, lambda i: (0, 0)),
            pl.BlockSpec(b3.shape, lambda i: (0,)),
        ],
        out_specs=pl.BlockSpec((bm,), lambda i: (i,)),
        out_shape=jax.ShapeDtypeStruct((b,), jnp.float32),
    )(u_vec, m_vec, w1u, w1m, b1, W2, b2, W3, b3)


def kernel(user_idx, movie_idx, user_table, movie_table, W1, b1, W2, b2, W3, b3):
    uidx = user_idx.astype(jnp.int32)
    midx = movie_idx.astype(jnp.int32)
    u_vec, m_vec = _sc_gather_rows(uidx, midx, user_table, movie_table)
    return _tc_mlp(u_vec, m_vec, W1, b1, W2, b2, W3, b3)
